# baseline (speedup 1.0000x reference)
import functools

import jax
import jax.numpy as jnp
from jax import lax
from jax.experimental import pallas as pl
from jax.experimental.pallas import tpu as pltpu
from jax.experimental.pallas import tpu_sc as plsc


def _make_sc_gather(n_tokens, d):
    nw = 32
    bpw = n_tokens // nw
    mesh = plsc.VectorSubcoreMesh(core_axis_name="c", subcore_axis_name="s")

    @functools.partial(
        pl.kernel,
        mesh=mesh,
        out_type=[
            jax.ShapeDtypeStruct((n_tokens, d), jnp.float32),
            jax.ShapeDtypeStruct((n_tokens, d), jnp.float32),
        ],
        scratch_types=[
            pltpu.VMEM((bpw,), jnp.int32),
            pltpu.VMEM((bpw,), jnp.int32),
            pltpu.VMEM((bpw,), jnp.int32),
            pltpu.VMEM((bpw, d), jnp.float32),
            pltpu.VMEM((bpw, d), jnp.float32),
            pltpu.SemaphoreType.DMA,
            pltpu.SemaphoreType.DMA,
        ],
    )
    def sc_gather(row_t_hbm, col_t_hbm, idx_hbm, row_out, col_out,
                  idx_v, rows_v, cols_v, rbuf, cbuf, sem_a, sem_b):
        wid = lax.axis_index("s") * 2 + lax.axis_index("c")
        base = wid * bpw
        pltpu.sync_copy(idx_hbm.at[pl.ds(base, bpw)], idx_v)
        for j in range(bpw // 16):
            v = idx_v[pl.ds(j * 16, 16)]
            rows_v[pl.ds(j * 16, 16)] = lax.shift_right_logical(v, 16)
            cols_v[pl.ds(j * 16, 16)] = lax.bitwise_and(v, 0xFFFF)
        cp_r = pltpu.async_copy(row_t_hbm.at[rows_v], rbuf, sem_a)
        cp_c = pltpu.async_copy(col_t_hbm.at[cols_v], cbuf, sem_b)
        cp_r.wait()
        cp_c.wait()
        pltpu.sync_copy(rbuf, row_out.at[pl.ds(base, bpw)])
        pltpu.sync_copy(cbuf, col_out.at[pl.ds(base, bpw)])

    return sc_gather


def _rotate_body(rg_ref, cg_ref, x_ref, o_ref, c_ref, sp_ref, sm_ref):
    d = x_ref.shape[-1]
    r = d // 4
    n = rg_ref.shape[0]

    @pl.when(pl.program_id(0) == 0)
    def _():
        rg = rg_ref[...]
        cg = cg_ref[...]
        c_ref[...] = jnp.concatenate([rg[:, :2 * r], cg[:, :2 * r]], axis=-1)
        s_full = jnp.concatenate([rg[:, 2 * r:], cg[:, 2 * r:]], axis=-1)
        lane = jax.lax.broadcasted_iota(jnp.int32, (n, d), 1)
        odd_grp = (lane & r).astype(jnp.bool_)
        zero = jnp.zeros((n, d), jnp.float32)
        sp_ref[...] = jnp.where(odd_grp, s_full, zero)
        sm_ref[...] = jnp.where(odd_grp, zero, s_full)

    nb = 64

    def body(ni, carry):
        s = pl.ds(ni * nb, nb)
        c8 = c_ref[s, :]
        sp8 = sp_ref[s, :]
        sm8 = sm_ref[s, :]
        for bi in range(x_ref.shape[0]):
            xb = x_ref[bi, s, :]
            xp = pltpu.roll(xb, r, 1)
            xm = pltpu.roll(xb, d - r, 1)
            o_ref[bi, s, :] = xb * c8 + xp * sp8 + xm * sm8
        return carry

    lax.fori_loop(0, n // nb, body, 0, unroll=2)


def kernel(x, idx, row_cs, col_cs):
    b, n, d = x.shape
    cos_r, sin_r = row_cs[..., 0], row_cs[..., 1]
    cos_c, sin_c = col_cs[..., 0], col_cs[..., 1]
    row_t = jnp.concatenate([cos_r, cos_r, -sin_r, sin_r], axis=1)
    col_t = jnp.concatenate([cos_c, cos_c, -sin_c, sin_c], axis=1)

    row_gat, col_gat = _make_sc_gather(n, d)(row_t, col_t, idx)

    bb = 16
    out = pl.pallas_call(
        _rotate_body,
        grid=(b // bb,),
        in_specs=[
            pl.BlockSpec((n, d), lambda i: (0, 0)),
            pl.BlockSpec((n, d), lambda i: (0, 0)),
            pl.BlockSpec((bb, n, d), lambda i: (i, 0, 0)),
        ],
        out_specs=pl.BlockSpec((bb, n, d), lambda i: (i, 0, 0)),
        out_shape=jax.ShapeDtypeStruct((b, n, d), jnp.float32),
        compiler_params=pltpu.CompilerParams(
            dimension_semantics=("parallel",)),
        scratch_shapes=[
            pltpu.VMEM((n, d), jnp.float32),
            pltpu.VMEM((n, d), jnp.float32),
            pltpu.VMEM((n, d), jnp.float32),
        ],
    )(row_gat, col_gat, x)
    return out


# speedup vs baseline: 1.0237x; 1.0237x over previous
import functools

import jax
import jax.numpy as jnp
from jax import lax
from jax.experimental import pallas as pl
from jax.experimental.pallas import tpu as pltpu
from jax.experimental.pallas import tpu_sc as plsc


def _make_sc_gather(n_tokens, d):
    nw = 32
    bpw = n_tokens // nw
    mesh = plsc.VectorSubcoreMesh(core_axis_name="c", subcore_axis_name="s")

    @functools.partial(
        pl.kernel,
        mesh=mesh,
        out_type=[
            jax.ShapeDtypeStruct((n_tokens, d), jnp.float32),
            jax.ShapeDtypeStruct((n_tokens, d), jnp.float32),
        ],
        scratch_types=[
            pltpu.VMEM((bpw,), jnp.int32),
            pltpu.VMEM((bpw,), jnp.int32),
            pltpu.VMEM((bpw,), jnp.int32),
            pltpu.VMEM((bpw, d), jnp.float32),
            pltpu.VMEM((bpw, d), jnp.float32),
            pltpu.SemaphoreType.DMA,
            pltpu.SemaphoreType.DMA,
        ],
    )
    def sc_gather(row_t_hbm, col_t_hbm, idx_hbm, row_out, col_out,
                  idx_v, rows_v, cols_v, rbuf, cbuf, sem_a, sem_b):
        wid = lax.axis_index("s") * 2 + lax.axis_index("c")
        base = wid * bpw
        pltpu.sync_copy(idx_hbm.at[pl.ds(base, bpw)], idx_v)
        for j in range(bpw // 16):
            v = idx_v[pl.ds(j * 16, 16)]
            rows_v[pl.ds(j * 16, 16)] = lax.shift_right_logical(v, 16)
            cols_v[pl.ds(j * 16, 16)] = lax.bitwise_and(v, 0xFFFF)
        cp_r = pltpu.async_copy(row_t_hbm.at[rows_v], rbuf, sem_a)
        cp_c = pltpu.async_copy(col_t_hbm.at[cols_v], cbuf, sem_b)
        cp_r.wait()
        cp_c.wait()
        pltpu.sync_copy(rbuf, row_out.at[pl.ds(base, bpw)])
        pltpu.sync_copy(cbuf, col_out.at[pl.ds(base, bpw)])

    return sc_gather


def _rotate_body(rg_ref, cg_ref, x_ref, o_ref, c_ref, sp_ref, sm_ref):
    d = x_ref.shape[-1]
    r = d // 4
    n = rg_ref.shape[0]

    @pl.when(pl.program_id(0) == 0)
    def _():
        rg = rg_ref[...]
        cg = cg_ref[...]
        c_ref[...] = jnp.concatenate([rg[:, :2 * r], cg[:, :2 * r]], axis=-1)
        s_full = jnp.concatenate([rg[:, 2 * r:], cg[:, 2 * r:]], axis=-1)
        lane = jax.lax.broadcasted_iota(jnp.int32, (n, d), 1)
        odd_grp = (lane & r).astype(jnp.bool_)
        zero = jnp.zeros((n, d), jnp.float32)
        sp_ref[...] = jnp.where(odd_grp, s_full, zero)
        sm_ref[...] = jnp.where(odd_grp, zero, s_full)

    xb = x_ref[...]
    xp = pltpu.roll(xb, r, 2)
    xm = pltpu.roll(xb, d - r, 2)
    o_ref[...] = (xb * c_ref[...][None] + xp * sp_ref[...][None]
                  + xm * sm_ref[...][None])


def kernel(x, idx, row_cs, col_cs):
    b, n, d = x.shape
    cos_r, sin_r = row_cs[..., 0], row_cs[..., 1]
    cos_c, sin_c = col_cs[..., 0], col_cs[..., 1]
    row_t = jnp.concatenate([cos_r, cos_r, -sin_r, sin_r], axis=1)
    col_t = jnp.concatenate([cos_c, cos_c, -sin_c, sin_c], axis=1)

    row_gat, col_gat = _make_sc_gather(n, d)(row_t, col_t, idx)

    bb = 16
    out = pl.pallas_call(
        _rotate_body,
        grid=(b // bb,),
        in_specs=[
            pl.BlockSpec((n, d), lambda i: (0, 0)),
            pl.BlockSpec((n, d), lambda i: (0, 0)),
            pl.BlockSpec((bb, n, d), lambda i: (i, 0, 0)),
        ],
        out_specs=pl.BlockSpec((bb, n, d), lambda i: (i, 0, 0)),
        out_shape=jax.ShapeDtypeStruct((b, n, d), jnp.float32),
        compiler_params=pltpu.CompilerParams(
            dimension_semantics=("parallel",)),
        scratch_shapes=[
            pltpu.VMEM((n, d), jnp.float32),
            pltpu.VMEM((n, d), jnp.float32),
            pltpu.VMEM((n, d), jnp.float32),
        ],
    )(row_gat, col_gat, x)
    return out


# speedup vs baseline: 1.0255x; 1.0017x over previous
import functools

import jax
import jax.numpy as jnp
from jax import lax
from jax.experimental import pallas as pl
from jax.experimental.pallas import tpu as pltpu
from jax.experimental.pallas import tpu_sc as plsc


def _make_sc_gather(n_tokens, d):
    nw = 32
    bpw = n_tokens // nw
    mesh = plsc.VectorSubcoreMesh(core_axis_name="c", subcore_axis_name="s")

    @functools.partial(
        pl.kernel,
        mesh=mesh,
        out_type=[
            jax.ShapeDtypeStruct((n_tokens, d), jnp.float32),
            jax.ShapeDtypeStruct((n_tokens, d), jnp.float32),
        ],
        scratch_types=[
            pltpu.VMEM((bpw,), jnp.int32),
            pltpu.VMEM((bpw,), jnp.int32),
            pltpu.VMEM((bpw,), jnp.int32),
            pltpu.VMEM((bpw, d), jnp.float32),
            pltpu.VMEM((bpw, d), jnp.float32),
            pltpu.SemaphoreType.DMA,
            pltpu.SemaphoreType.DMA,
        ],
    )
    def sc_gather(row_t_hbm, col_t_hbm, idx_hbm, row_out, col_out,
                  idx_v, rows_v, cols_v, rbuf, cbuf, sem_a, sem_b):
        wid = lax.axis_index("s") * 2 + lax.axis_index("c")
        base = wid * bpw
        pltpu.sync_copy(idx_hbm.at[pl.ds(base, bpw)], idx_v)
        for j in range(bpw // 16):
            v = idx_v[pl.ds(j * 16, 16)]
            rows_v[pl.ds(j * 16, 16)] = lax.shift_right_logical(v, 16)
            cols_v[pl.ds(j * 16, 16)] = lax.bitwise_and(v, 0xFFFF)
        cp_r = pltpu.async_copy(row_t_hbm.at[rows_v], rbuf, sem_a)
        cp_c = pltpu.async_copy(col_t_hbm.at[cols_v], cbuf, sem_b)
        cp_r.wait()
        cp_c.wait()
        pltpu.sync_copy(rbuf, row_out.at[pl.ds(base, bpw)])
        pltpu.sync_copy(cbuf, col_out.at[pl.ds(base, bpw)])

    return sc_gather


def _rotate_body(rg_ref, cg_ref, x_ref, o_ref, c_ref, sp_ref):
    d = x_ref.shape[-1]
    r = d // 4
    n = rg_ref.shape[0]

    @pl.when(pl.program_id(0) == 0)
    def _():
        rg = rg_ref[...]
        cg = cg_ref[...]
        c_ref[...] = jnp.concatenate([rg[:, :2 * r], cg[:, :2 * r]], axis=-1)
        sp_ref[...] = jnp.concatenate([rg[:, 2 * r:], cg[:, 2 * r:]], axis=-1)

    xb = x_ref[...]
    bb = xb.shape[0]
    lane = jax.lax.broadcasted_iota(jnp.int32, (bb, n, d), 2)
    odd_grp = (lane & r) != 0
    xp = pltpu.roll(xb, r, 2)
    xm = pltpu.roll(xb, d - r, 2)
    xsw = jnp.where(odd_grp, xp, xm)
    o_ref[...] = xb * c_ref[...][None] + xsw * sp_ref[...][None]


def kernel(x, idx, row_cs, col_cs):
    b, n, d = x.shape
    cos_r, sin_r = row_cs[..., 0], row_cs[..., 1]
    cos_c, sin_c = col_cs[..., 0], col_cs[..., 1]
    row_t = jnp.concatenate([cos_r, cos_r, -sin_r, sin_r], axis=1)
    col_t = jnp.concatenate([cos_c, cos_c, -sin_c, sin_c], axis=1)

    row_gat, col_gat = _make_sc_gather(n, d)(row_t, col_t, idx)

    bb = 16
    out = pl.pallas_call(
        _rotate_body,
        grid=(b // bb,),
        in_specs=[
            pl.BlockSpec((n, d), lambda i: (0, 0)),
            pl.BlockSpec((n, d), lambda i: (0, 0)),
            pl.BlockSpec((bb, n, d), lambda i: (i, 0, 0)),
        ],
        out_specs=pl.BlockSpec((bb, n, d), lambda i: (i, 0, 0)),
        out_shape=jax.ShapeDtypeStruct((b, n, d), jnp.float32),
        compiler_params=pltpu.CompilerParams(
            dimension_semantics=("parallel",)),
        scratch_shapes=[
            pltpu.VMEM((n, d), jnp.float32),
            pltpu.VMEM((n, d), jnp.float32),
        ],
    )(row_gat, col_gat, x)
    return out
